# baseline (speedup 1.0000x reference)
import functools

import jax
import jax.numpy as jnp
from jax import lax
from jax.experimental import pallas as pl
from jax.experimental.pallas import tpu as pltpu
from jax.experimental.pallas import tpu_sc as plsc

_NBUF = 4
_TC_SEQS = 64


def _halves(L):
  h0 = (L // 2) // 8 * 8
  return h0, L - h0


@functools.lru_cache(maxsize=None)
def _build_gather(B, L, V, D, NC, NS):
  NW = NC * NS
  assert B % NW == 0 and (B // NW) % _NBUF == 0, (B, NW)
  spw = B // NW
  H0, HB = _halves(L)
  assert H0 % 8 == 0 and HB % 8 == 0 and HB <= 128 and D == 64

  mesh = plsc.VectorSubcoreMesh(
      core_axis_name="c", subcore_axis_name="s",
      num_cores=NC, num_subcores=NS)

  @functools.partial(
      pl.kernel,
      out_type=jax.ShapeDtypeStruct((B * HB, 2 * D), jnp.float32),
      mesh=mesh,
      scratch_types=(
          [pltpu.VMEM((spw, L), jnp.int32)]
          + [pltpu.VMEM((L, D), jnp.float32) for _ in range(_NBUF)]
          + [pltpu.SemaphoreType.DMA for _ in range(2 * _NBUF)]
      ),
      compiler_params=pltpu.CompilerParams(use_tc_tiling_on_sc=False),
  )
  def k(x_hbm, tok_hbm, out_hbm, idx_all, *bufsem):
    rows = bufsem[:_NBUF]
    gsems = bufsem[_NBUF:2 * _NBUF]
    ssems = bufsem[2 * _NBUF:]
    wid = lax.axis_index("s") * NC + lax.axis_index("c")
    base_seq = wid * spw
    pltpu.sync_copy(x_hbm.at[pl.ds(base_seq, spw)], idx_all)

    def issue_gather(i, b):
      pltpu.async_copy(
          tok_hbm.at[idx_all.at[i, pl.ds(0, H0)]],
          rows[b].at[pl.ds(0, H0)], gsems[b])
      pltpu.async_copy(
          tok_hbm.at[idx_all.at[i, pl.ds(H0, HB)]],
          rows[b].at[pl.ds(H0, HB)], gsems[b])

    def wait_gather(b):
      pltpu.make_async_copy(tok_hbm.at[pl.ds(0, L)], rows[b], gsems[b]).wait()

    def out_slab(i):
      return out_hbm.at[pl.ds((base_seq + i) * HB, HB)]

    def issue_scatter(i, b):
      pltpu.async_copy(rows[b].at[pl.ds(0, H0)],
                       out_slab(i).at[pl.ds(0, H0), pl.ds(0, D)], ssems[b])
      pltpu.async_copy(rows[b].at[pl.ds(H0, HB)],
                       out_slab(i).at[:, pl.ds(D, D)], ssems[b])

    def wait_scatter(b):
      pltpu.make_async_copy(rows[b].at[pl.ds(0, H0)],
                            out_slab(0).at[pl.ds(0, H0), pl.ds(0, D)],
                            ssems[b]).wait()
      pltpu.make_async_copy(rows[b].at[pl.ds(H0, HB)],
                            out_slab(0).at[:, pl.ds(D, D)], ssems[b]).wait()

    for j in range(_NBUF - 1):
      issue_gather(j, j)

    def body(p, carry):
      for b in range(_NBUF):
        i = p * _NBUF + b
        wait_gather(b)

        nb = (b + _NBUF - 1) % _NBUF

        @pl.when(i + _NBUF - 1 < spw)
        def _prefetch():
          @pl.when(i + _NBUF - 1 >= _NBUF)
          def _wait_prev_scatter():
            wait_scatter(nb)
          issue_gather(i + _NBUF - 1, nb)

        issue_scatter(i, b)
      return carry

    lax.fori_loop(0, spw // _NBUF, body, 0)
    for b in range(_NBUF):
      wait_scatter(b)

  return k


@functools.lru_cache(maxsize=None)
def _build_add(B, L, D):
  H0, HB = _halves(L)
  blk_in = _TC_SEQS * HB
  blk_out = _TC_SEQS * L
  grid = B // _TC_SEQS

  def body(i_ref, p_ref, o_ref):
    z = i_ref[...].reshape(_TC_SEQS, HB, 2 * D) + p_ref[...][None]
    o_ref[...] = jnp.concatenate([z[:, :H0, :D], z[:, :, D:]], axis=1)

  return pl.pallas_call(
      body,
      grid=(grid,),
      in_specs=[
          pl.BlockSpec((blk_in, 2 * D), lambda i: (i, 0)),
          pl.BlockSpec((HB, 2 * D), lambda i: (0, 0)),
      ],
      out_specs=pl.BlockSpec((_TC_SEQS, L, D), lambda i: (i, 0, 0)),
      out_shape=jax.ShapeDtypeStruct((B, L, D), jnp.float32),
  )


def kernel(x, token_table, pos_table):
  B, L = x.shape
  V, D = token_table.shape
  H0, HB = _halves(L)
  try:
    info = plsc.get_sparse_core_info()
    NC, NS = info.num_cores, info.num_subcores
  except Exception:
    NC, NS = 2, 16
  gathered = _build_gather(B, L, V, D, NC, NS)(
      x.astype(jnp.int32), token_table)
  left = jnp.concatenate(
      [pos_table[:H0], jnp.zeros((HB - H0, D), jnp.float32)], axis=0)
  pos2 = jnp.concatenate([left, pos_table[H0:]], axis=1)
  return _build_add(B, L, D)(gathered, pos2)


# speedup vs baseline: 1.1867x; 1.1867x over previous
import functools

import jax
import jax.numpy as jnp
from jax import lax
from jax.experimental import pallas as pl
from jax.experimental.pallas import tpu as pltpu
from jax.experimental.pallas import tpu_sc as plsc

_NBUF = 4
_TC_SEQS = 128


def _halves(L):
  h0 = (L // 2) // 8 * 8
  return h0, L - h0


@functools.lru_cache(maxsize=None)
def _build_gather(B, L, V, D, NC, NS):
  NW = NC * NS
  assert B % NW == 0 and (B // NW) % _NBUF == 0, (B, NW)
  spw = B // NW
  H0, HB = _halves(L)
  assert H0 % 8 == 0 and HB % 8 == 0 and HB <= 128 and D == 64

  mesh = plsc.VectorSubcoreMesh(
      core_axis_name="c", subcore_axis_name="s",
      num_cores=NC, num_subcores=NS)

  @functools.partial(
      pl.kernel,
      out_type=jax.ShapeDtypeStruct((B * HB, 2 * D), jnp.float32),
      mesh=mesh,
      scratch_types=(
          [pltpu.VMEM((spw, L), jnp.int32)]
          + [pltpu.VMEM((L, D), jnp.float32) for _ in range(_NBUF)]
          + [pltpu.SemaphoreType.DMA for _ in range(2 * _NBUF)]
      ),
      compiler_params=pltpu.CompilerParams(use_tc_tiling_on_sc=False),
  )
  def k(x_hbm, tok_hbm, out_hbm, idx_all, *bufsem):
    rows = bufsem[:_NBUF]
    gsems = bufsem[_NBUF:2 * _NBUF]
    ssems = bufsem[2 * _NBUF:]
    wid = lax.axis_index("s") * NC + lax.axis_index("c")
    base_seq = wid * spw
    pltpu.sync_copy(x_hbm.at[pl.ds(base_seq, spw)], idx_all)

    def issue_gather(i, b):
      pltpu.async_copy(
          tok_hbm.at[idx_all.at[i, pl.ds(0, H0)]],
          rows[b].at[pl.ds(0, H0)], gsems[b])
      pltpu.async_copy(
          tok_hbm.at[idx_all.at[i, pl.ds(H0, HB)]],
          rows[b].at[pl.ds(H0, HB)], gsems[b])

    def wait_gather(b):
      pltpu.make_async_copy(tok_hbm.at[pl.ds(0, L)], rows[b], gsems[b]).wait()

    def out_slab(i):
      return out_hbm.at[pl.ds((base_seq + i) * HB, HB)]

    def issue_scatter(i, b):
      pltpu.async_copy(rows[b].at[pl.ds(0, H0)],
                       out_slab(i).at[pl.ds(0, H0), pl.ds(0, D)], ssems[b])
      pltpu.async_copy(rows[b].at[pl.ds(H0, HB)],
                       out_slab(i).at[:, pl.ds(D, D)], ssems[b])

    def wait_scatter(b):
      pltpu.make_async_copy(rows[b].at[pl.ds(0, H0)],
                            out_slab(0).at[pl.ds(0, H0), pl.ds(0, D)],
                            ssems[b]).wait()
      pltpu.make_async_copy(rows[b].at[pl.ds(H0, HB)],
                            out_slab(0).at[:, pl.ds(D, D)], ssems[b]).wait()

    for j in range(_NBUF - 1):
      issue_gather(j, j)

    def body(p, carry):
      for b in range(_NBUF):
        i = p * _NBUF + b
        wait_gather(b)

        nb = (b + _NBUF - 1) % _NBUF

        @pl.when(i + _NBUF - 1 < spw)
        def _prefetch():
          @pl.when(i + _NBUF - 1 >= _NBUF)
          def _wait_prev_scatter():
            wait_scatter(nb)
          issue_gather(i + _NBUF - 1, nb)

        issue_scatter(i, b)
      return carry

    lax.fori_loop(0, spw // _NBUF, body, 0)
    for b in range(_NBUF):
      wait_scatter(b)

  return k


@functools.lru_cache(maxsize=None)
def _build_add(B, L, D):
  H0, HB = _halves(L)
  blk_in = _TC_SEQS * HB
  blk_out = _TC_SEQS * L
  grid = B // _TC_SEQS

  def body(i_ref, p_ref, o_ref):
    z = i_ref[...].reshape(_TC_SEQS, HB, 2 * D) + p_ref[...][None]
    o_ref[...] = jnp.concatenate(
        [z[:, :H0, :D], z[:, :, D:]], axis=1).reshape(blk_out, D)

  return pl.pallas_call(
      body,
      grid=(grid,),
      in_specs=[
          pl.BlockSpec((blk_in, 2 * D), lambda i: (i, 0)),
          pl.BlockSpec((HB, 2 * D), lambda i: (0, 0)),
      ],
      out_specs=pl.BlockSpec((blk_out, D), lambda i: (i, 0)),
      out_shape=jax.ShapeDtypeStruct((B * L, D), jnp.float32),
  )


def kernel(x, token_table, pos_table):
  B, L = x.shape
  V, D = token_table.shape
  H0, HB = _halves(L)
  try:
    info = plsc.get_sparse_core_info()
    NC, NS = info.num_cores, info.num_subcores
  except Exception:
    NC, NS = 2, 16
  gathered = _build_gather(B, L, V, D, NC, NS)(
      x.astype(jnp.int32), token_table)
  left = jnp.concatenate(
      [pos_table[:H0], jnp.zeros((HB - H0, D), jnp.float32)], axis=0)
  pos2 = jnp.concatenate([left, pos_table[H0:]], axis=1)
  return _build_add(B, L, D)(gathered, pos2).reshape(B, L, D)
